# baseline (speedup 1.0000x reference)
import jax
import jax.numpy as jnp
from jax import lax
from jax.experimental import pallas as pl
from jax.experimental.pallas import tpu as pltpu
from jax.experimental.pallas import tpu_sc as plsc

DIM = 64
N_EDGES = 819200

NC = 2
NS = 16
LANES = 16
NW = NC * NS
E_PER_W = N_EDGES // NW
CHUNK = 64
N_HALF = 1
E_PER_H = E_PER_W // N_HALF
N_CHUNKS = E_PER_H // CHUNK
PAD = 17
NBUF = 8


def _body(xp_hbm, xt_hbm, eidx_hbm, out_hbm,
          idx_p, idx_t, rows_p, rows_t, out_all,
          sem_p0, sem_p1, sem_p2, sem_p3, sem_p4, sem_p5, sem_p6, sem_p7,
          sem_t0, sem_t1, sem_t2, sem_t3, sem_t4, sem_t5, sem_t6, sem_t7):
    wid = lax.axis_index("s") * NC + lax.axis_index("c")
    base = wid * E_PER_W
    sems_p = [sem_p0, sem_p1, sem_p2, sem_p3, sem_p4, sem_p5, sem_p6,
              sem_p7][:NBUF]
    sems_t = [sem_t0, sem_t1, sem_t2, sem_t3, sem_t4, sem_t5, sem_t6,
              sem_t7][:NBUF]

    iota = lax.iota(jnp.int32, LANES)

    def fire(k, b):
        pltpu.async_copy(
            xp_hbm.at[idx_p.at[pl.ds(k * CHUNK, CHUNK)]],
            rows_p.at[b], sems_p[b])
        pltpu.async_copy(
            xt_hbm.at[idx_t.at[pl.ds(k * CHUNK, CHUNK)]],
            rows_t.at[b], sems_t[b])

    for h in range(N_HALF):
        hbase = base + h * E_PER_H
        pltpu.sync_copy(eidx_hbm.at[0, pl.ds(hbase, E_PER_H)], idx_p)
        pltpu.sync_copy(eidx_hbm.at[1, pl.ds(hbase, E_PER_H)], idx_t)

        for j in range(NBUF - 1):
            fire(j, j)

        @pl.loop(0, N_CHUNKS, step=NBUF)
        def chunk_pair(c):
            for b in range(NBUF):
                k = c + b
                nb = (b + NBUF - 1) % NBUF

                @pl.when(k + NBUF - 1 < N_CHUNKS)
                def _():
                    fire(k + NBUF - 1, nb)

                pltpu.make_async_copy(
                    xp_hbm.at[idx_p.at[pl.ds(k * CHUNK, CHUNK)]],
                    rows_p.at[b], sems_p[b]).wait()
                pltpu.make_async_copy(
                    xt_hbm.at[idx_t.at[pl.ds(k * CHUNK, CHUNK)]],
                    rows_t.at[b], sems_t[b]).wait()

                rp = rows_p.at[b]
                rt = rows_t.at[b]

                @plsc.parallel_loop(0, CHUNK // LANES, unroll=1)
                def group_body(g):
                    accs = []
                    for j in range(LANES):
                        e = g * LANES + j
                        prod = (rp[e, pl.ds(0, 2 * LANES)]
                                * rt[e, pl.ds(0, 2 * LANES)])
                        prod += (rp[e, pl.ds(2 * LANES, 2 * LANES)]
                                 * rt[e, pl.ds(2 * LANES, 2 * LANES)])
                        lo, hi = plsc.unpack(
                            prod, format=plsc.PackFormat.INTERLEAVED)
                        accs.append(lo + hi)

                    dnums = lax.GatherDimensionNumbers(
                        offset_dims=(), collapsed_slice_dims=(0,),
                        start_index_map=(0,))
                    vecs = accs
                    d = 1
                    while len(vecs) > 1:
                        mask = (iota & d) == 0
                        perm = (iota ^ d)[:, None]
                        nxt = []
                        for i in range(len(vecs) // 2):
                            a, bb = vecs[2 * i], vecs[2 * i + 1]
                            x = jnp.where(mask, a, bb)
                            y = jnp.where(mask, bb, a)
                            yp = lax.gather(
                                y, perm, dnums, (1,),
                                unique_indices=True,
                                mode=lax.GatherScatterMode.PROMISE_IN_BOUNDS)
                            nxt.append(x + yp)
                        vecs = nxt
                        d *= 2
                    out_all[pl.ds(h * E_PER_H + k * CHUNK + g * LANES,
                                  LANES)] = vecs[0]

    pltpu.sync_copy(out_all, out_hbm.at[pl.ds(base, E_PER_W)])


@jax.jit
def kernel(x_playlist, x_track, edge_label_index):
    eidx = edge_label_index.astype(jnp.int32)
    xp = x_playlist.astype(jnp.bfloat16)
    xt = x_track.astype(jnp.bfloat16)

    mesh = plsc.VectorSubcoreMesh(core_axis_name="c", subcore_axis_name="s")
    run = pl.kernel(
        _body,
        out_type=jax.ShapeDtypeStruct((N_EDGES,), jnp.float32),
        mesh=mesh,
        compiler_params=pltpu.CompilerParams(
            needs_layout_passes=False, use_tc_tiling_on_sc=False),
        scratch_types=[
            pltpu.VMEM((E_PER_H,), jnp.int32),
            pltpu.VMEM((E_PER_H,), jnp.int32),
            pltpu.VMEM((NBUF, CHUNK, DIM), jnp.bfloat16),
            pltpu.VMEM((NBUF, CHUNK, DIM), jnp.bfloat16),
            pltpu.VMEM((E_PER_W,), jnp.float32),
            pltpu.SemaphoreType.DMA,
            pltpu.SemaphoreType.DMA,
            pltpu.SemaphoreType.DMA,
            pltpu.SemaphoreType.DMA,
            pltpu.SemaphoreType.DMA,
            pltpu.SemaphoreType.DMA,
            pltpu.SemaphoreType.DMA,
            pltpu.SemaphoreType.DMA,
            pltpu.SemaphoreType.DMA,
            pltpu.SemaphoreType.DMA,
            pltpu.SemaphoreType.DMA,
            pltpu.SemaphoreType.DMA,
            pltpu.SemaphoreType.DMA,
            pltpu.SemaphoreType.DMA,
            pltpu.SemaphoreType.DMA,
            pltpu.SemaphoreType.DMA,
        ],
    )
    return run(xp, xt, eidx)


# speedup vs baseline: 1.2632x; 1.2632x over previous
import jax
import jax.numpy as jnp
from jax import lax
from jax.experimental import pallas as pl
from jax.experimental.pallas import tpu as pltpu
from jax.experimental.pallas import tpu_sc as plsc

DIM = 64
N_EDGES = 819200

NC = 2
NS = 16
LANES = 16
NW = NC * NS
E_PER_W = N_EDGES // NW
CHUNK = 128
N_HALF = 1
E_PER_H = E_PER_W // N_HALF
N_CHUNKS = E_PER_H // CHUNK
PAD = 17
NBUF = 5


def _body(xp_hbm, xt_hbm, eidx_hbm, out_hbm,
          idx_p, idx_t, rows_p, rows_t, out_all,
          sem_p0, sem_p1, sem_p2, sem_p3, sem_p4, sem_p5, sem_p6, sem_p7,
          sem_t0, sem_t1, sem_t2, sem_t3, sem_t4, sem_t5, sem_t6, sem_t7):
    wid = lax.axis_index("s") * NC + lax.axis_index("c")
    base = wid * E_PER_W
    sems_p = [sem_p0, sem_p1, sem_p2, sem_p3, sem_p4, sem_p5, sem_p6,
              sem_p7][:NBUF]
    sems_t = [sem_t0, sem_t1, sem_t2, sem_t3, sem_t4, sem_t5, sem_t6,
              sem_t7][:NBUF]

    iota = lax.iota(jnp.int32, LANES)

    def fire(k, b):
        pltpu.async_copy(
            xp_hbm.at[idx_p.at[pl.ds(k * CHUNK, CHUNK)]],
            rows_p.at[b], sems_p[b])
        pltpu.async_copy(
            xt_hbm.at[idx_t.at[pl.ds(k * CHUNK, CHUNK)]],
            rows_t.at[b], sems_t[b])

    for h in range(N_HALF):
        hbase = base + h * E_PER_H
        pltpu.sync_copy(eidx_hbm.at[0, pl.ds(hbase, E_PER_H)], idx_p)
        pltpu.sync_copy(eidx_hbm.at[1, pl.ds(hbase, E_PER_H)], idx_t)

        for j in range(NBUF - 1):
            fire(j, j)

        @pl.loop(0, N_CHUNKS, step=NBUF)
        def chunk_pair(c):
            for b in range(NBUF):
                k = c + b
                nb = (b + NBUF - 1) % NBUF

                @pl.when(k + NBUF - 1 < N_CHUNKS)
                def _():
                    fire(k + NBUF - 1, nb)

                pltpu.make_async_copy(
                    xp_hbm.at[idx_p.at[pl.ds(k * CHUNK, CHUNK)]],
                    rows_p.at[b], sems_p[b]).wait()
                pltpu.make_async_copy(
                    xt_hbm.at[idx_t.at[pl.ds(k * CHUNK, CHUNK)]],
                    rows_t.at[b], sems_t[b]).wait()

                rp = rows_p.at[b]
                rt = rows_t.at[b]

                @plsc.parallel_loop(0, CHUNK // LANES, unroll=1)
                def group_body(g):
                    accs = []
                    for j in range(LANES):
                        e = g * LANES + j
                        prod = (rp[e, pl.ds(0, 2 * LANES)]
                                * rt[e, pl.ds(0, 2 * LANES)])
                        prod += (rp[e, pl.ds(2 * LANES, 2 * LANES)]
                                 * rt[e, pl.ds(2 * LANES, 2 * LANES)])
                        lo, hi = plsc.unpack(
                            prod, format=plsc.PackFormat.INTERLEAVED)
                        accs.append(lo + hi)

                    dnums = lax.GatherDimensionNumbers(
                        offset_dims=(), collapsed_slice_dims=(0,),
                        start_index_map=(0,))
                    vecs = accs
                    d = 1
                    while len(vecs) > 1:
                        mask = (iota & d) == 0
                        perm = (iota ^ d)[:, None]
                        nxt = []
                        for i in range(len(vecs) // 2):
                            a, bb = vecs[2 * i], vecs[2 * i + 1]
                            x = jnp.where(mask, a, bb)
                            y = jnp.where(mask, bb, a)
                            yp = lax.gather(
                                y, perm, dnums, (1,),
                                unique_indices=True,
                                mode=lax.GatherScatterMode.PROMISE_IN_BOUNDS)
                            nxt.append(x + yp)
                        vecs = nxt
                        d *= 2
                    out_all[pl.ds(h * E_PER_H + k * CHUNK + g * LANES,
                                  LANES)] = vecs[0]

    pltpu.sync_copy(out_all, out_hbm.at[pl.ds(base, E_PER_W)])


@jax.jit
def kernel(x_playlist, x_track, edge_label_index):
    eidx = edge_label_index.astype(jnp.int32)
    xp = x_playlist.astype(jnp.bfloat16)
    xt = x_track.astype(jnp.bfloat16)

    mesh = plsc.VectorSubcoreMesh(core_axis_name="c", subcore_axis_name="s")
    run = pl.kernel(
        _body,
        out_type=jax.ShapeDtypeStruct((N_EDGES,), jnp.float32),
        mesh=mesh,
        compiler_params=pltpu.CompilerParams(
            needs_layout_passes=False, use_tc_tiling_on_sc=False),
        scratch_types=[
            pltpu.VMEM((E_PER_H,), jnp.int32),
            pltpu.VMEM((E_PER_H,), jnp.int32),
            pltpu.VMEM((NBUF, CHUNK, DIM), jnp.bfloat16),
            pltpu.VMEM((NBUF, CHUNK, DIM), jnp.bfloat16),
            pltpu.VMEM((E_PER_W,), jnp.float32),
            pltpu.SemaphoreType.DMA,
            pltpu.SemaphoreType.DMA,
            pltpu.SemaphoreType.DMA,
            pltpu.SemaphoreType.DMA,
            pltpu.SemaphoreType.DMA,
            pltpu.SemaphoreType.DMA,
            pltpu.SemaphoreType.DMA,
            pltpu.SemaphoreType.DMA,
            pltpu.SemaphoreType.DMA,
            pltpu.SemaphoreType.DMA,
            pltpu.SemaphoreType.DMA,
            pltpu.SemaphoreType.DMA,
            pltpu.SemaphoreType.DMA,
            pltpu.SemaphoreType.DMA,
            pltpu.SemaphoreType.DMA,
            pltpu.SemaphoreType.DMA,
        ],
    )
    return run(xp, xt, eidx)


# speedup vs baseline: 1.2671x; 1.0032x over previous
import jax
import jax.numpy as jnp
from jax import lax
from jax.experimental import pallas as pl
from jax.experimental.pallas import tpu as pltpu
from jax.experimental.pallas import tpu_sc as plsc

DIM = 64
N_EDGES = 819200

NC = 2
NS = 16
LANES = 16
NW = NC * NS
E_PER_W = N_EDGES // NW
CHUNK = 128
N_HALF = 1
E_PER_H = E_PER_W // N_HALF
N_CHUNKS = E_PER_H // CHUNK
PAD = 17
NBUF = 4


def _body(xp_hbm, xt_hbm, eidx_hbm, out_hbm,
          idx_p, idx_t, rows_p, rows_t, out_all,
          sem_p0, sem_p1, sem_p2, sem_p3, sem_p4, sem_p5, sem_p6, sem_p7,
          sem_t0, sem_t1, sem_t2, sem_t3, sem_t4, sem_t5, sem_t6, sem_t7):
    wid = lax.axis_index("s") * NC + lax.axis_index("c")
    base = wid * E_PER_W
    sems_p = [sem_p0, sem_p1, sem_p2, sem_p3, sem_p4, sem_p5, sem_p6,
              sem_p7][:NBUF]
    sems_t = [sem_t0, sem_t1, sem_t2, sem_t3, sem_t4, sem_t5, sem_t6,
              sem_t7][:NBUF]

    iota = lax.iota(jnp.int32, LANES)

    def fire(k, b):
        pltpu.async_copy(
            xp_hbm.at[idx_p.at[pl.ds(k * CHUNK, CHUNK)]],
            rows_p.at[b], sems_p[b])
        pltpu.async_copy(
            xt_hbm.at[idx_t.at[pl.ds(k * CHUNK, CHUNK)]],
            rows_t.at[b], sems_t[b])

    for h in range(N_HALF):
        hbase = base + h * E_PER_H
        pltpu.sync_copy(eidx_hbm.at[0, pl.ds(hbase, E_PER_H)], idx_p)
        pltpu.sync_copy(eidx_hbm.at[1, pl.ds(hbase, E_PER_H)], idx_t)

        for j in range(NBUF - 1):
            fire(j, j)

        @pl.loop(0, N_CHUNKS, step=NBUF)
        def chunk_pair(c):
            for b in range(NBUF):
                k = c + b
                nb = (b + NBUF - 1) % NBUF

                @pl.when(k + NBUF - 1 < N_CHUNKS)
                def _():
                    fire(k + NBUF - 1, nb)

                pltpu.make_async_copy(
                    xp_hbm.at[idx_p.at[pl.ds(k * CHUNK, CHUNK)]],
                    rows_p.at[b], sems_p[b]).wait()
                pltpu.make_async_copy(
                    xt_hbm.at[idx_t.at[pl.ds(k * CHUNK, CHUNK)]],
                    rows_t.at[b], sems_t[b]).wait()

                rp = rows_p.at[b]
                rt = rows_t.at[b]

                @plsc.parallel_loop(0, CHUNK // LANES, unroll=1)
                def group_body(g):
                    accs = []
                    for j in range(LANES):
                        e = g * LANES + j
                        prod = (rp[e, pl.ds(0, 2 * LANES)]
                                * rt[e, pl.ds(0, 2 * LANES)])
                        prod += (rp[e, pl.ds(2 * LANES, 2 * LANES)]
                                 * rt[e, pl.ds(2 * LANES, 2 * LANES)])
                        lo, hi = plsc.unpack(
                            prod, format=plsc.PackFormat.INTERLEAVED)
                        accs.append(lo + hi)

                    dnums = lax.GatherDimensionNumbers(
                        offset_dims=(), collapsed_slice_dims=(0,),
                        start_index_map=(0,))
                    vecs = accs
                    d = 1
                    while len(vecs) > 1:
                        mask = (iota & d) == 0
                        perm = (iota ^ d)[:, None]
                        nxt = []
                        for i in range(len(vecs) // 2):
                            a, bb = vecs[2 * i], vecs[2 * i + 1]
                            x = jnp.where(mask, a, bb)
                            y = jnp.where(mask, bb, a)
                            yp = lax.gather(
                                y, perm, dnums, (1,),
                                unique_indices=True,
                                mode=lax.GatherScatterMode.PROMISE_IN_BOUNDS)
                            nxt.append(x + yp)
                        vecs = nxt
                        d *= 2
                    out_all[pl.ds(h * E_PER_H + k * CHUNK + g * LANES,
                                  LANES)] = vecs[0]

    pltpu.sync_copy(out_all, out_hbm.at[pl.ds(base, E_PER_W)])


@jax.jit
def kernel(x_playlist, x_track, edge_label_index):
    eidx = edge_label_index.astype(jnp.int32)
    xp = x_playlist.astype(jnp.bfloat16)
    xt = x_track.astype(jnp.bfloat16)

    mesh = plsc.VectorSubcoreMesh(core_axis_name="c", subcore_axis_name="s")
    run = pl.kernel(
        _body,
        out_type=jax.ShapeDtypeStruct((N_EDGES,), jnp.float32),
        mesh=mesh,
        compiler_params=pltpu.CompilerParams(
            needs_layout_passes=False, use_tc_tiling_on_sc=False),
        scratch_types=[
            pltpu.VMEM((E_PER_H,), jnp.int32),
            pltpu.VMEM((E_PER_H,), jnp.int32),
            pltpu.VMEM((NBUF, CHUNK, DIM), jnp.bfloat16),
            pltpu.VMEM((NBUF, CHUNK, DIM), jnp.bfloat16),
            pltpu.VMEM((E_PER_W,), jnp.float32),
            pltpu.SemaphoreType.DMA,
            pltpu.SemaphoreType.DMA,
            pltpu.SemaphoreType.DMA,
            pltpu.SemaphoreType.DMA,
            pltpu.SemaphoreType.DMA,
            pltpu.SemaphoreType.DMA,
            pltpu.SemaphoreType.DMA,
            pltpu.SemaphoreType.DMA,
            pltpu.SemaphoreType.DMA,
            pltpu.SemaphoreType.DMA,
            pltpu.SemaphoreType.DMA,
            pltpu.SemaphoreType.DMA,
            pltpu.SemaphoreType.DMA,
            pltpu.SemaphoreType.DMA,
            pltpu.SemaphoreType.DMA,
            pltpu.SemaphoreType.DMA,
        ],
    )
    return run(xp, xt, eidx)


# speedup vs baseline: 1.2979x; 1.0243x over previous
import jax
import jax.numpy as jnp
from jax import lax
from jax.experimental import pallas as pl
from jax.experimental.pallas import tpu as pltpu
from jax.experimental.pallas import tpu_sc as plsc

DIM = 64
N_EDGES = 819200

NC = 2
NS = 16
LANES = 16
NW = NC * NS
E_PER_W = N_EDGES // NW
CHUNK = 128
N_HALF = 1
E_PER_H = E_PER_W // N_HALF
N_CHUNKS = E_PER_H // CHUNK
PAD = 17
NBUF = 4


def _body(xp_hbm, xt_hbm, eidx_hbm, out_hbm,
          idx_p, idx_t, rows_p, rows_t, out_all,
          sem_p0, sem_p1, sem_p2, sem_p3, sem_p4, sem_p5, sem_p6, sem_p7,
          sem_t0, sem_t1, sem_t2, sem_t3, sem_t4, sem_t5, sem_t6, sem_t7):
    wid = lax.axis_index("s") * NC + lax.axis_index("c")
    base = wid * E_PER_W
    sems_p = [sem_p0, sem_p1, sem_p2, sem_p3, sem_p4, sem_p5, sem_p6,
              sem_p7][:NBUF]
    sems_t = [sem_t0, sem_t1, sem_t2, sem_t3, sem_t4, sem_t5, sem_t6,
              sem_t7][:NBUF]

    iota = lax.iota(jnp.int32, LANES)

    def fire(k, b):
        pltpu.async_copy(
            xp_hbm.at[idx_p.at[pl.ds(k * CHUNK, CHUNK)]],
            rows_p.at[b], sems_p[b])
        pltpu.async_copy(
            xt_hbm.at[idx_t.at[pl.ds(k * CHUNK, CHUNK)]],
            rows_t.at[b], sems_t[b])

    for h in range(N_HALF):
        hbase = base + h * E_PER_H
        pltpu.sync_copy(eidx_hbm.at[0, pl.ds(hbase, E_PER_H)], idx_p)
        pltpu.sync_copy(eidx_hbm.at[1, pl.ds(hbase, E_PER_H)], idx_t)

        for j in range(NBUF - 1):
            fire(j, j)

        @pl.loop(0, N_CHUNKS, step=NBUF)
        def chunk_pair(c):
            for b in range(NBUF):
                k = c + b
                nb = (b + NBUF - 1) % NBUF

                @pl.when(k + NBUF - 1 < N_CHUNKS)
                def _():
                    fire(k + NBUF - 1, nb)

                pltpu.make_async_copy(
                    xp_hbm.at[idx_p.at[pl.ds(k * CHUNK, CHUNK)]],
                    rows_p.at[b], sems_p[b]).wait()
                pltpu.make_async_copy(
                    xt_hbm.at[idx_t.at[pl.ds(k * CHUNK, CHUNK)]],
                    rows_t.at[b], sems_t[b]).wait()

                rp = rows_p.at[b]
                rt = rows_t.at[b]

                @plsc.parallel_loop(0, CHUNK // LANES, unroll=2)
                def group_body(g):
                    accs = []
                    for j in range(LANES):
                        e = g * LANES + j
                        prod = (rp[e, pl.ds(0, 2 * LANES)]
                                * rt[e, pl.ds(0, 2 * LANES)])
                        prod += (rp[e, pl.ds(2 * LANES, 2 * LANES)]
                                 * rt[e, pl.ds(2 * LANES, 2 * LANES)])
                        lo, hi = plsc.unpack(
                            prod, format=plsc.PackFormat.INTERLEAVED)
                        accs.append(lo + hi)

                    dnums = lax.GatherDimensionNumbers(
                        offset_dims=(), collapsed_slice_dims=(0,),
                        start_index_map=(0,))
                    vecs = accs
                    d = 1
                    while len(vecs) > 1:
                        mask = (iota & d) == 0
                        perm = (iota ^ d)[:, None]
                        nxt = []
                        for i in range(len(vecs) // 2):
                            a, bb = vecs[2 * i], vecs[2 * i + 1]
                            x = jnp.where(mask, a, bb)
                            y = jnp.where(mask, bb, a)
                            yp = lax.gather(
                                y, perm, dnums, (1,),
                                unique_indices=True,
                                mode=lax.GatherScatterMode.PROMISE_IN_BOUNDS)
                            nxt.append(x + yp)
                        vecs = nxt
                        d *= 2
                    out_all[pl.ds(h * E_PER_H + k * CHUNK + g * LANES,
                                  LANES)] = vecs[0]

    pltpu.sync_copy(out_all, out_hbm.at[pl.ds(base, E_PER_W)])


@jax.jit
def kernel(x_playlist, x_track, edge_label_index):
    eidx = edge_label_index.astype(jnp.int32)
    xp = x_playlist.astype(jnp.bfloat16)
    xt = x_track.astype(jnp.bfloat16)

    mesh = plsc.VectorSubcoreMesh(core_axis_name="c", subcore_axis_name="s")
    run = pl.kernel(
        _body,
        out_type=jax.ShapeDtypeStruct((N_EDGES,), jnp.float32),
        mesh=mesh,
        compiler_params=pltpu.CompilerParams(
            needs_layout_passes=False, use_tc_tiling_on_sc=False),
        scratch_types=[
            pltpu.VMEM((E_PER_H,), jnp.int32),
            pltpu.VMEM((E_PER_H,), jnp.int32),
            pltpu.VMEM((NBUF, CHUNK, DIM), jnp.bfloat16),
            pltpu.VMEM((NBUF, CHUNK, DIM), jnp.bfloat16),
            pltpu.VMEM((E_PER_W,), jnp.float32),
            pltpu.SemaphoreType.DMA,
            pltpu.SemaphoreType.DMA,
            pltpu.SemaphoreType.DMA,
            pltpu.SemaphoreType.DMA,
            pltpu.SemaphoreType.DMA,
            pltpu.SemaphoreType.DMA,
            pltpu.SemaphoreType.DMA,
            pltpu.SemaphoreType.DMA,
            pltpu.SemaphoreType.DMA,
            pltpu.SemaphoreType.DMA,
            pltpu.SemaphoreType.DMA,
            pltpu.SemaphoreType.DMA,
            pltpu.SemaphoreType.DMA,
            pltpu.SemaphoreType.DMA,
            pltpu.SemaphoreType.DMA,
            pltpu.SemaphoreType.DMA,
        ],
    )
    return run(xp, xt, eidx)
